# Initial kernel scaffold; baseline (speedup 1.0000x reference)
import jax
import jax.numpy as jnp
from jax.experimental import pallas as pl


def kernel(x, edge_index, W1, b1, g1, be1, m1, v1, W2, b2, g2, be2, m2, v2, Wf, bf):
    raise NotImplementedError("write your pallas kernel here")


# speedup vs baseline: 6.0843x; 6.0843x over previous
import functools

import jax
import jax.numpy as jnp
from jax import lax
from jax.experimental import pallas as pl
from jax.experimental.pallas import tpu as pltpu
from jax.experimental.pallas import tpu_sc as plsc

N = 10000
E = 320000
D = 128

NC = 2
NS = 16
NW = NC * NS

K = 80
HALF = 5120
HRPT = HALF // NS
TRASH = HALF
ACCR = HALF + 8

EPS = E // NS
NCHS = EPS // K

_mesh = plsc.VectorSubcoreMesh(core_axis_name="c", subcore_axis_name="s")


def _make_deg_kernel(interpret=False):
    return functools.partial(
        pl.kernel,
        out_type=jax.ShapeDtypeStruct((NC, HALF, D), jnp.float32),
        mesh=_mesh,
        interpret=interpret,
        scratch_types=[
            pltpu.VMEM((K,), jnp.int32),
            pltpu.VMEM((K, D), jnp.float32),
            pltpu.VMEM((HRPT, D), jnp.float32),
            pltpu.VMEM_SHARED((ACCR, D), jnp.float32),
        ],
    )(_deg_body)


def _deg_body(dst_hbm, out_hbm, didx_v, ones_v, zb_v, acc_sh):
    c = lax.axis_index("c")
    s = lax.axis_index("s")
    lo = c * HALF

    def _fill(i, carry):
        for q in range(D // 16):
            ones_v[i, pl.ds(q * 16, 16)] = jnp.ones((16,), jnp.float32)
        return carry

    lax.fori_loop(0, K, _fill, 0)

    def _zfill(i, carry):
        for q in range(D // 16):
            zb_v[i, pl.ds(q * 16, 16)] = jnp.zeros((16,), jnp.float32)
        return carry

    lax.fori_loop(0, HRPT, _zfill, 0)
    pltpu.sync_copy(zb_v, acc_sh.at[pl.ds(s * HRPT, HRPT)])
    pltpu.sync_copy(zb_v.at[pl.ds(0, 8)], acc_sh.at[pl.ds(TRASH, 8)])
    plsc.subcore_barrier()

    def _body(j, carry):
        base = s * EPS + j * K
        pltpu.sync_copy(dst_hbm.at[pl.ds(base, K)], didx_v)
        for q in range(K // 16):
            d = didx_v[pl.ds(q * 16, 16)]
            local = d - lo
            ok = (local >= 0) & (local < HALF)
            didx_v[pl.ds(q * 16, 16)] = jnp.where(ok, local, TRASH)
        pltpu.sync_copy(ones_v, acc_sh.at[didx_v], add=True)
        return carry

    lax.fori_loop(0, NCHS, _body, 0)
    plsc.subcore_barrier()
    pltpu.sync_copy(acc_sh.at[pl.ds(s * HRPT, HRPT)], zb_v)
    pltpu.sync_copy(zb_v, out_hbm.at[c, pl.ds(s * HRPT, HRPT)])


def _make_agg_kernel(interpret=False):
    return functools.partial(
        pl.kernel,
        out_type=jax.ShapeDtypeStruct((NC, HALF, D), jnp.float32),
        mesh=_mesh,
        interpret=interpret,
        scratch_types=[
            pltpu.VMEM((K,), jnp.int32),
            pltpu.VMEM((K,), jnp.int32),
            pltpu.VMEM((K, D), jnp.float32),
            pltpu.VMEM((HRPT, D), jnp.float32),
            pltpu.VMEM_SHARED((ACCR, D), jnp.float32),
            pltpu.SemaphoreType.DMA,
        ],
    )(_agg_body)


def _agg_body(z_hbm, src_hbm, dst_hbm, out_hbm,
              sidx_v, didx_v, mbuf_v, zb_v, acc_sh, sem):
    c = lax.axis_index("c")
    s = lax.axis_index("s")
    lo = c * HALF

    def _zfill(i, carry):
        for q in range(D // 16):
            zb_v[i, pl.ds(q * 16, 16)] = jnp.zeros((16,), jnp.float32)
        return carry

    lax.fori_loop(0, HRPT, _zfill, 0)
    pltpu.sync_copy(zb_v, acc_sh.at[pl.ds(s * HRPT, HRPT)])
    pltpu.sync_copy(zb_v.at[pl.ds(0, 8)], acc_sh.at[pl.ds(TRASH, 8)])
    plsc.subcore_barrier()

    def _body(j, carry):
        base = s * EPS + j * K
        pltpu.sync_copy(src_hbm.at[pl.ds(base, K)], sidx_v)
        pltpu.sync_copy(dst_hbm.at[pl.ds(base, K)], didx_v)
        for q in range(K // 16):
            d = didx_v[pl.ds(q * 16, 16)]
            local = d - lo
            ok = (local >= 0) & (local < HALF)
            didx_v[pl.ds(q * 16, 16)] = jnp.where(ok, local, TRASH)
        pltpu.async_copy(z_hbm.at[sidx_v], mbuf_v, sem).wait()
        pltpu.sync_copy(mbuf_v, acc_sh.at[didx_v], add=True)
        return carry

    lax.fori_loop(0, NCHS, _body, 0)
    plsc.subcore_barrier()
    pltpu.sync_copy(acc_sh.at[pl.ds(s * HRPT, HRPT)], zb_v)
    pltpu.sync_copy(zb_v, out_hbm.at[c, pl.ds(s * HRPT, HRPT)])


_deg_kernel = _make_deg_kernel()
_agg_kernel = _make_agg_kernel()


R = 2000


def _dinv(deg_ref):
    return lax.rsqrt(deg_ref[:, 0:1] + 1.0)


def _bn_relu(x, g_ref, be_ref, m_ref, v_ref):
    h = (x - m_ref[...]) * lax.rsqrt(v_ref[...] + 1e-5) * g_ref[...] + be_ref[...]
    return jnp.maximum(h, 0.0)


def _pre_body(x_ref, deg_ref, g_ref, be_ref, m_ref, v_ref, w_ref, zs_ref):
    h = _bn_relu(x_ref[...], g_ref, be_ref, m_ref, v_ref)
    z = jnp.dot(h, w_ref[...], preferred_element_type=jnp.float32,
                precision=lax.Precision.HIGHEST)
    zs_ref[...] = z * _dinv(deg_ref)


def _mid_body(x_ref, agg_ref, zs_ref, deg_ref, b1_ref,
              g_ref, be_ref, m_ref, v_ref, w_ref, x1_ref, zs2_ref):
    dinv = _dinv(deg_ref)
    x1 = (x_ref[...] + dinv * (agg_ref[...] + zs_ref[...])
          + b1_ref[...])
    x1_ref[...] = x1
    h = _bn_relu(x1, g_ref, be_ref, m_ref, v_ref)
    z = jnp.dot(h, w_ref[...], preferred_element_type=jnp.float32,
                precision=lax.Precision.HIGHEST)
    zs2_ref[...] = z * dinv


def _post_body(x1_ref, agg_ref, zs2_ref, deg_ref, b2_ref, wf_ref, bf_ref,
               out_ref):
    dinv = _dinv(deg_ref)
    x2 = (x1_ref[...] + dinv * (agg_ref[...] + zs2_ref[...])
          + b2_ref[...])
    out_ref[...] = jnp.dot(x2, wf_ref[...], preferred_element_type=jnp.float32,
                           precision=lax.Precision.HIGHEST) + bf_ref[...]


_ROWS = pl.BlockSpec((R, D), lambda i: (i, 0))
_PARM = pl.BlockSpec((1, D), lambda i: (0, 0))
_WMAT = pl.BlockSpec((D, D), lambda i: (0, 0))


def _pre(x, deg, g, be, m, v, w):
    return pl.pallas_call(
        _pre_body,
        grid=(N // R,),
        in_specs=[_ROWS, _ROWS, _PARM, _PARM, _PARM, _PARM, _WMAT],
        out_specs=_ROWS,
        out_shape=jax.ShapeDtypeStruct((N, D), jnp.float32),
    )(x, deg, g, be, m, v, w)


def _mid(x, agg, zs, deg, b1, g, be, m, v, w):
    return pl.pallas_call(
        _mid_body,
        grid=(N // R,),
        in_specs=[_ROWS, _ROWS, _ROWS, _ROWS, _PARM,
                  _PARM, _PARM, _PARM, _PARM, _WMAT],
        out_specs=(_ROWS, _ROWS),
        out_shape=(jax.ShapeDtypeStruct((N, D), jnp.float32),
                   jax.ShapeDtypeStruct((N, D), jnp.float32)),
    )(x, agg, zs, deg, b1, g, be, m, v, w)


def _post(x1, agg, zs2, deg, b2, wf, bf):
    return pl.pallas_call(
        _post_body,
        grid=(N // R,),
        in_specs=[_ROWS, _ROWS, _ROWS, _ROWS, _PARM, _WMAT, _PARM],
        out_specs=_ROWS,
        out_shape=jax.ShapeDtypeStruct((N, D), jnp.float32),
    )(x1, agg, zs2, deg, b2, wf, bf)


def kernel(x, edge_index, W1, b1, g1, be1, m1, v1,
           W2, b2, g2, be2, m2, v2, Wf, bf):
    src = edge_index[0]
    dst = edge_index[1]
    r = lambda p: p.reshape(1, D)

    cat = lambda a: jnp.concatenate([a[0], a[1, 0:(N - HALF)]], axis=0)

    deg = cat(_deg_kernel(dst))
    zs1 = _pre(x, deg, r(g1), r(be1), r(m1), r(v1), W1)
    agg1 = cat(_agg_kernel(zs1, src, dst))
    x1, zs2 = _mid(x, agg1, zs1, deg, r(b1),
                   r(g2), r(be2), r(m2), r(v2), W2)
    agg2 = cat(_agg_kernel(zs2, src, dst))
    return _post(x1, agg2, zs2, deg, r(b2), Wf, r(bf))


# speedup vs baseline: 11.2090x; 1.8423x over previous
import functools

import jax
import jax.numpy as jnp
from jax import lax
from jax.experimental import pallas as pl
from jax.experimental.pallas import tpu as pltpu
from jax.experimental.pallas import tpu_sc as plsc

N = 10000
E = 320000
D = 128

NC = 2
NS = 16
NW = NC * NS

K = 80
HALF = 5120
HRPT = HALF // NS
TRASH = HALF
ACCR = HALF + 8

ZR = 80
EPS = E // NS
NCHS = EPS // K

_mesh = plsc.VectorSubcoreMesh(core_axis_name="c", subcore_axis_name="s")


def _make_deg_kernel(interpret=False):
    return functools.partial(
        pl.kernel,
        out_type=jax.ShapeDtypeStruct((NC, HALF, D), jnp.float32),
        mesh=_mesh,
        interpret=interpret,
        scratch_types=[
            pltpu.VMEM((K,), jnp.int32),
            pltpu.VMEM((K, D), jnp.float32),
            pltpu.VMEM((HRPT, D), jnp.float32),
            pltpu.VMEM_SHARED((ACCR, D), jnp.float32),
        ],
    )(_deg_body)


def _deg_body(dst_hbm, out_hbm, didx_v, ones_v, zb_v, acc_sh):
    c = lax.axis_index("c")
    s = lax.axis_index("s")
    lo = c * HALF

    def _fill(i, carry):
        for q in range(D // 16):
            ones_v[i, pl.ds(q * 16, 16)] = jnp.ones((16,), jnp.float32)
        return carry

    lax.fori_loop(0, K, _fill, 0)

    def _zfill(i, carry):
        for q in range(D // 16):
            zb_v[i, pl.ds(q * 16, 16)] = jnp.zeros((16,), jnp.float32)
        return carry

    lax.fori_loop(0, HRPT, _zfill, 0)
    pltpu.sync_copy(zb_v, acc_sh.at[pl.ds(s * HRPT, HRPT)])
    pltpu.sync_copy(zb_v.at[pl.ds(0, 8)], acc_sh.at[pl.ds(TRASH, 8)])
    plsc.subcore_barrier()

    def _body(j, carry):
        base = s * EPS + j * K
        pltpu.sync_copy(dst_hbm.at[pl.ds(base, K)], didx_v)
        for q in range(K // 16):
            d = didx_v[pl.ds(q * 16, 16)]
            local = d - lo
            ok = (local >= 0) & (local < HALF)
            didx_v[pl.ds(q * 16, 16)] = jnp.where(ok, local, TRASH)
        pltpu.sync_copy(ones_v, acc_sh.at[didx_v], add=True)
        return carry

    lax.fori_loop(0, NCHS, _body, 0)
    plsc.subcore_barrier()
    pltpu.sync_copy(acc_sh.at[pl.ds(s * HRPT, HRPT)], zb_v)
    pltpu.sync_copy(zb_v, out_hbm.at[c, pl.ds(s * HRPT, HRPT)])


def _make_agg_kernel(interpret=False):
    return functools.partial(
        pl.kernel,
        out_type=jax.ShapeDtypeStruct((NC, HALF, D), jnp.float32),
        mesh=_mesh,
        interpret=interpret,
        scratch_types=[
            pltpu.VMEM((EPS,), jnp.int32),
            pltpu.VMEM((EPS,), jnp.int32),
            pltpu.VMEM((K,), jnp.int32),
            pltpu.VMEM((2, K, D), jnp.float32),
            pltpu.VMEM((ZR, D), jnp.float32),
            pltpu.VMEM_SHARED((ACCR, D), jnp.float32),
            pltpu.SemaphoreType.DMA((2,)),
        ],
    )(_agg_body)


def _agg_body(z_hbm, src_hbm, dst_hbm, out_hbm,
              sidx_v, didx_v, rdx_v, mbuf_v, zb_v, acc_sh, sems):
    c = lax.axis_index("c")
    s = lax.axis_index("s")
    lo = c * HALF

    def _zfill(i, carry):
        for q in range(D // 16):
            zb_v[i, pl.ds(q * 16, 16)] = jnp.zeros((16,), jnp.float32)
        return carry

    lax.fori_loop(0, ZR, _zfill, 0)
    for t in range(HRPT // ZR):
        pltpu.sync_copy(zb_v, acc_sh.at[pl.ds(s * HRPT + t * ZR, ZR)])
    pltpu.sync_copy(zb_v.at[pl.ds(0, 8)], acc_sh.at[pl.ds(TRASH, 8)])
    pltpu.sync_copy(src_hbm.at[pl.ds(s * EPS, EPS)], sidx_v)
    pltpu.sync_copy(dst_hbm.at[pl.ds(s * EPS, EPS)], didx_v)
    plsc.subcore_barrier()

    def _gather(j, p):
        pltpu.async_copy(z_hbm.at[sidx_v.at[pl.ds(j * K, K)]],
                         mbuf_v.at[p], sems.at[p])

    _gather(0, 0)

    def _body(j, carry):
        p = lax.rem(j, 2)

        @pl.when(j + 1 < NCHS)
        def _():
            _gather(j + 1, 1 - p)

        for q in range(K // 16):
            d = didx_v[pl.ds(j * K + q * 16, 16)]
            local = d - lo
            ok = (local >= 0) & (local < HALF)
            rdx_v[pl.ds(q * 16, 16)] = jnp.where(ok, local, TRASH)
        pltpu.make_async_copy(z_hbm.at[sidx_v.at[pl.ds(j * K, K)]],
                              mbuf_v.at[p], sems.at[p]).wait()
        pltpu.sync_copy(mbuf_v.at[p], acc_sh.at[rdx_v], add=True)
        return carry

    lax.fori_loop(0, NCHS, _body, 0)
    plsc.subcore_barrier()
    for t in range(HRPT // ZR):
        pltpu.sync_copy(acc_sh.at[pl.ds(s * HRPT + t * ZR, ZR)], zb_v)
        pltpu.sync_copy(zb_v, out_hbm.at[c, pl.ds(s * HRPT + t * ZR, ZR)])


_deg_kernel = _make_deg_kernel()
_agg_kernel = _make_agg_kernel()


R = 2000


def _dinv(deg_ref):
    return lax.rsqrt(deg_ref[:, 0:1] + 1.0)


def _bn_relu(x, g_ref, be_ref, m_ref, v_ref):
    h = (x - m_ref[...]) * lax.rsqrt(v_ref[...] + 1e-5) * g_ref[...] + be_ref[...]
    return jnp.maximum(h, 0.0)


def _pre_body(x_ref, deg_ref, g_ref, be_ref, m_ref, v_ref, w_ref, zs_ref):
    h = _bn_relu(x_ref[...], g_ref, be_ref, m_ref, v_ref)
    z = jnp.dot(h, w_ref[...], preferred_element_type=jnp.float32,
                precision=lax.Precision.HIGHEST)
    zs_ref[...] = z * _dinv(deg_ref)


def _mid_body(x_ref, agg_ref, zs_ref, deg_ref, b1_ref,
              g_ref, be_ref, m_ref, v_ref, w_ref, x1_ref, zs2_ref):
    dinv = _dinv(deg_ref)
    x1 = (x_ref[...] + dinv * (agg_ref[...] + zs_ref[...])
          + b1_ref[...])
    x1_ref[...] = x1
    h = _bn_relu(x1, g_ref, be_ref, m_ref, v_ref)
    z = jnp.dot(h, w_ref[...], preferred_element_type=jnp.float32,
                precision=lax.Precision.HIGHEST)
    zs2_ref[...] = z * dinv


def _post_body(x1_ref, agg_ref, zs2_ref, deg_ref, b2_ref, wf_ref, bf_ref,
               out_ref):
    dinv = _dinv(deg_ref)
    x2 = (x1_ref[...] + dinv * (agg_ref[...] + zs2_ref[...])
          + b2_ref[...])
    out_ref[...] = jnp.dot(x2, wf_ref[...], preferred_element_type=jnp.float32,
                           precision=lax.Precision.HIGHEST) + bf_ref[...]


_ROWS = pl.BlockSpec((R, D), lambda i: (i, 0))
_PARM = pl.BlockSpec((1, D), lambda i: (0, 0))
_WMAT = pl.BlockSpec((D, D), lambda i: (0, 0))


def _pre(x, deg, g, be, m, v, w):
    return pl.pallas_call(
        _pre_body,
        grid=(N // R,),
        in_specs=[_ROWS, _ROWS, _PARM, _PARM, _PARM, _PARM, _WMAT],
        out_specs=_ROWS,
        out_shape=jax.ShapeDtypeStruct((N, D), jnp.float32),
    )(x, deg, g, be, m, v, w)


def _mid(x, agg, zs, deg, b1, g, be, m, v, w):
    return pl.pallas_call(
        _mid_body,
        grid=(N // R,),
        in_specs=[_ROWS, _ROWS, _ROWS, _ROWS, _PARM,
                  _PARM, _PARM, _PARM, _PARM, _WMAT],
        out_specs=(_ROWS, _ROWS),
        out_shape=(jax.ShapeDtypeStruct((N, D), jnp.float32),
                   jax.ShapeDtypeStruct((N, D), jnp.float32)),
    )(x, agg, zs, deg, b1, g, be, m, v, w)


def _post(x1, agg, zs2, deg, b2, wf, bf):
    return pl.pallas_call(
        _post_body,
        grid=(N // R,),
        in_specs=[_ROWS, _ROWS, _ROWS, _ROWS, _PARM, _WMAT, _PARM],
        out_specs=_ROWS,
        out_shape=jax.ShapeDtypeStruct((N, D), jnp.float32),
    )(x1, agg, zs2, deg, b2, wf, bf)


def kernel(x, edge_index, W1, b1, g1, be1, m1, v1,
           W2, b2, g2, be2, m2, v2, Wf, bf):
    src = edge_index[0]
    dst = edge_index[1]
    r = lambda p: p.reshape(1, D)

    cat = lambda a: jnp.concatenate([a[0], a[1, 0:(N - HALF)]], axis=0)

    deg = cat(_deg_kernel(dst))
    zs1 = _pre(x, deg, r(g1), r(be1), r(m1), r(v1), W1)
    agg1 = cat(_agg_kernel(zs1, src, dst))
    x1, zs2 = _mid(x, agg1, zs1, deg, r(b1),
                   r(g2), r(be2), r(m2), r(v2), W2)
    agg2 = cat(_agg_kernel(zs2, src, dst))
    return _post(x1, agg2, zs2, deg, r(b2), Wf, r(bf))


# speedup vs baseline: 11.4777x; 1.0240x over previous
import functools

import jax
import jax.numpy as jnp
from jax import lax
from jax.experimental import pallas as pl
from jax.experimental.pallas import tpu as pltpu
from jax.experimental.pallas import tpu_sc as plsc

N = 10000
E = 320000
D = 128

NC = 2
NS = 16
NW = NC * NS

K = 80
HALF = 5120
HRPT = HALF // NS
TRASH = HALF
ACCR = HALF + 8

ZR = 80
EPS = E // NS
NCHS = EPS // K

_mesh = plsc.VectorSubcoreMesh(core_axis_name="c", subcore_axis_name="s")


def _make_deg_kernel(interpret=False):
    return functools.partial(
        pl.kernel,
        out_type=jax.ShapeDtypeStruct((NC, HALF, D), jnp.float32),
        mesh=_mesh,
        interpret=interpret,
        scratch_types=[
            pltpu.VMEM((EPS,), jnp.int32),
            pltpu.VMEM((2, K), jnp.int32),
            pltpu.VMEM((K, D), jnp.float32),
            pltpu.VMEM((ZR, D), jnp.float32),
            pltpu.VMEM_SHARED((ACCR, D), jnp.float32),
            pltpu.SemaphoreType.DMA((2,)),
        ],
    )(_deg_body)


def _deg_body(dst_hbm, out_hbm, didx_v, rdx_v, ones_v, zb_v, acc_sh, sems):
    c = lax.axis_index("c")
    s = lax.axis_index("s")
    lo = c * HALF

    def _fill(i, carry):
        for q in range(D // 16):
            ones_v[i, pl.ds(q * 16, 16)] = jnp.ones((16,), jnp.float32)
        return carry

    lax.fori_loop(0, K, _fill, 0)

    def _zfill(i, carry):
        for q in range(D // 16):
            zb_v[i, pl.ds(q * 16, 16)] = jnp.zeros((16,), jnp.float32)
        return carry

    lax.fori_loop(0, ZR, _zfill, 0)
    for t in range(HRPT // ZR):
        pltpu.sync_copy(zb_v, acc_sh.at[pl.ds(s * HRPT + t * ZR, ZR)])
    pltpu.sync_copy(zb_v.at[pl.ds(0, 8)], acc_sh.at[pl.ds(TRASH, 8)])
    pltpu.sync_copy(dst_hbm.at[pl.ds(s * EPS, EPS)], didx_v)
    plsc.subcore_barrier()

    def _body(j, carry):
        p = lax.rem(j, 2)

        @pl.when(j >= 2)
        def _():
            pltpu.make_async_copy(ones_v, acc_sh.at[rdx_v.at[p]],
                                  sems.at[p]).wait()

        for q in range(K // 16):
            d = didx_v[pl.ds(j * K + q * 16, 16)]
            local = d - lo
            ok = (local >= 0) & (local < HALF)
            rdx_v[p, pl.ds(q * 16, 16)] = jnp.where(ok, local, TRASH)
        pltpu.async_copy(ones_v, acc_sh.at[rdx_v.at[p]], sems.at[p],
                         add=True)
        return carry

    lax.fori_loop(0, NCHS, _body, 0)
    for p in range(2):
        pltpu.make_async_copy(ones_v, acc_sh.at[rdx_v.at[p]],
                              sems.at[p]).wait()
    plsc.subcore_barrier()
    for t in range(HRPT // ZR):
        pltpu.sync_copy(acc_sh.at[pl.ds(s * HRPT + t * ZR, ZR)], zb_v)
        pltpu.sync_copy(zb_v, out_hbm.at[c, pl.ds(s * HRPT + t * ZR, ZR)])


def _make_agg_kernel(interpret=False):
    return functools.partial(
        pl.kernel,
        out_type=jax.ShapeDtypeStruct((NC, HALF, D), jnp.float32),
        mesh=_mesh,
        interpret=interpret,
        scratch_types=[
            pltpu.VMEM((EPS,), jnp.int32),
            pltpu.VMEM((EPS,), jnp.int32),
            pltpu.VMEM((K,), jnp.int32),
            pltpu.VMEM((2, K, D), jnp.float32),
            pltpu.VMEM((ZR, D), jnp.float32),
            pltpu.VMEM_SHARED((ACCR, D), jnp.float32),
            pltpu.SemaphoreType.DMA((2,)),
        ],
    )(_agg_body)


def _agg_body(z_hbm, src_hbm, dst_hbm, out_hbm,
              sidx_v, didx_v, rdx_v, mbuf_v, zb_v, acc_sh, sems):
    c = lax.axis_index("c")
    s = lax.axis_index("s")
    lo = c * HALF

    def _zfill(i, carry):
        for q in range(D // 16):
            zb_v[i, pl.ds(q * 16, 16)] = jnp.zeros((16,), jnp.float32)
        return carry

    lax.fori_loop(0, ZR, _zfill, 0)
    for t in range(HRPT // ZR):
        pltpu.sync_copy(zb_v, acc_sh.at[pl.ds(s * HRPT + t * ZR, ZR)])
    pltpu.sync_copy(zb_v.at[pl.ds(0, 8)], acc_sh.at[pl.ds(TRASH, 8)])
    pltpu.sync_copy(src_hbm.at[pl.ds(s * EPS, EPS)], sidx_v)
    pltpu.sync_copy(dst_hbm.at[pl.ds(s * EPS, EPS)], didx_v)
    plsc.subcore_barrier()

    def _gather(j, p):
        pltpu.async_copy(z_hbm.at[sidx_v.at[pl.ds(j * K, K)]],
                         mbuf_v.at[p], sems.at[p])

    _gather(0, 0)

    def _body(j, carry):
        p = lax.rem(j, 2)

        @pl.when(j + 1 < NCHS)
        def _():
            _gather(j + 1, 1 - p)

        for q in range(K // 16):
            d = didx_v[pl.ds(j * K + q * 16, 16)]
            local = d - lo
            ok = (local >= 0) & (local < HALF)
            rdx_v[pl.ds(q * 16, 16)] = jnp.where(ok, local, TRASH)
        pltpu.make_async_copy(z_hbm.at[sidx_v.at[pl.ds(j * K, K)]],
                              mbuf_v.at[p], sems.at[p]).wait()
        pltpu.sync_copy(mbuf_v.at[p], acc_sh.at[rdx_v], add=True)
        return carry

    lax.fori_loop(0, NCHS, _body, 0)
    plsc.subcore_barrier()
    for t in range(HRPT // ZR):
        pltpu.sync_copy(acc_sh.at[pl.ds(s * HRPT + t * ZR, ZR)], zb_v)
        pltpu.sync_copy(zb_v, out_hbm.at[c, pl.ds(s * HRPT + t * ZR, ZR)])


_deg_kernel = _make_deg_kernel()
_agg_kernel = _make_agg_kernel()


R = 2000


def _dinv(deg_ref):
    return lax.rsqrt(deg_ref[:, 0:1] + 1.0)


def _bn_relu(x, g_ref, be_ref, m_ref, v_ref):
    h = (x - m_ref[...]) * lax.rsqrt(v_ref[...] + 1e-5) * g_ref[...] + be_ref[...]
    return jnp.maximum(h, 0.0)


def _pre_body(x_ref, deg_ref, g_ref, be_ref, m_ref, v_ref, w_ref, zs_ref):
    h = _bn_relu(x_ref[...], g_ref, be_ref, m_ref, v_ref)
    z = jnp.dot(h, w_ref[...], preferred_element_type=jnp.float32,
                precision=lax.Precision.HIGHEST)
    zs_ref[...] = z * _dinv(deg_ref)


def _mid_body(x_ref, agg_ref, zs_ref, deg_ref, b1_ref,
              g_ref, be_ref, m_ref, v_ref, w_ref, x1_ref, zs2_ref):
    dinv = _dinv(deg_ref)
    x1 = (x_ref[...] + dinv * (agg_ref[...] + zs_ref[...])
          + b1_ref[...])
    x1_ref[...] = x1
    h = _bn_relu(x1, g_ref, be_ref, m_ref, v_ref)
    z = jnp.dot(h, w_ref[...], preferred_element_type=jnp.float32,
                precision=lax.Precision.HIGHEST)
    zs2_ref[...] = z * dinv


def _post_body(x1_ref, agg_ref, zs2_ref, deg_ref, b2_ref, wf_ref, bf_ref,
               out_ref):
    dinv = _dinv(deg_ref)
    x2 = (x1_ref[...] + dinv * (agg_ref[...] + zs2_ref[...])
          + b2_ref[...])
    out_ref[...] = jnp.dot(x2, wf_ref[...], preferred_element_type=jnp.float32,
                           precision=lax.Precision.HIGHEST) + bf_ref[...]


_ROWS = pl.BlockSpec((R, D), lambda i: (i, 0))
_PARM = pl.BlockSpec((1, D), lambda i: (0, 0))
_WMAT = pl.BlockSpec((D, D), lambda i: (0, 0))


def _pre(x, deg, g, be, m, v, w):
    return pl.pallas_call(
        _pre_body,
        grid=(N // R,),
        in_specs=[_ROWS, _ROWS, _PARM, _PARM, _PARM, _PARM, _WMAT],
        out_specs=_ROWS,
        out_shape=jax.ShapeDtypeStruct((N, D), jnp.float32),
    )(x, deg, g, be, m, v, w)


def _mid(x, agg, zs, deg, b1, g, be, m, v, w):
    return pl.pallas_call(
        _mid_body,
        grid=(N // R,),
        in_specs=[_ROWS, _ROWS, _ROWS, _ROWS, _PARM,
                  _PARM, _PARM, _PARM, _PARM, _WMAT],
        out_specs=(_ROWS, _ROWS),
        out_shape=(jax.ShapeDtypeStruct((N, D), jnp.float32),
                   jax.ShapeDtypeStruct((N, D), jnp.float32)),
    )(x, agg, zs, deg, b1, g, be, m, v, w)


def _post(x1, agg, zs2, deg, b2, wf, bf):
    return pl.pallas_call(
        _post_body,
        grid=(N // R,),
        in_specs=[_ROWS, _ROWS, _ROWS, _ROWS, _PARM, _WMAT, _PARM],
        out_specs=_ROWS,
        out_shape=jax.ShapeDtypeStruct((N, D), jnp.float32),
    )(x1, agg, zs2, deg, b2, wf, bf)


def kernel(x, edge_index, W1, b1, g1, be1, m1, v1,
           W2, b2, g2, be2, m2, v2, Wf, bf):
    src = edge_index[0]
    dst = edge_index[1]
    r = lambda p: p.reshape(1, D)

    cat = lambda a: jnp.concatenate([a[0], a[1, 0:(N - HALF)]], axis=0)

    deg = cat(_deg_kernel(dst))
    zs1 = _pre(x, deg, r(g1), r(be1), r(m1), r(v1), W1)
    agg1 = cat(_agg_kernel(zs1, src, dst))
    x1, zs2 = _mid(x, agg1, zs1, deg, r(b1),
                   r(g2), r(be2), r(m2), r(v2), W2)
    agg2 = cat(_agg_kernel(zs2, src, dst))
    return _post(x1, agg2, zs2, deg, r(b2), Wf, r(bf))
